# baseline (speedup 1.0000x reference)
import functools

import jax
import jax.numpy as jnp
from jax import lax
from jax.experimental import pallas as pl
from jax.experimental.pallas import tpu as pltpu
from jax.experimental.pallas import tpu_sc as plsc

K = 8
PAD_COORD = 1.0e18
BIG = 3.0e38

QB = 256
VPAD = 5120

NW = 32
CPG = 8
QT = 8
SPLIT = 0


def _round_bf16_f32(x):
    u = lax.bitcast_convert_type(x, jnp.uint32)
    r = (u + jnp.uint32(0x7FFF) + ((u >> 16) & jnp.uint32(1))) & jnp.uint32(
        0xFFFF0000
    )
    return lax.bitcast_convert_type(r, jnp.float32)


def _knn_block(mb_ref, m2_ref, vb_ref, v2_ref, out_ref):
    mb = mb_ref[:, :].astype(jnp.float32)
    mx = mb[:, 0:1]
    my = mb[:, 1:2]
    mz = mb[:, 2:3]
    m2 = m2_ref[:, :]
    vb = vb_ref[:, :].astype(jnp.float32)
    vx = vb[0:1, :]
    vy = vb[1:2, :]
    vz = vb[2:3, :]
    v2 = v2_ref[:, :]
    mv = (mx * vx + my * vy) + mz * vz
    d2 = (m2 - 2.0 * mv) + v2
    iot = jax.lax.broadcasted_iota(jnp.int32, (QB, VPAD), 1)
    cols = []
    for _ in range(K):
        mn = jnp.min(d2, axis=1, keepdims=True)
        am = jnp.min(
            jnp.where(d2 == mn, iot, jnp.int32(2**30)), axis=1, keepdims=True
        )
        cols.append(am)
        d2 = jnp.where(iot == am, jnp.float32(jnp.inf), d2)
    out_ref[:, :] = jnp.concatenate(cols, axis=1)


def _tc_call(means, vertices):
    q = means.shape[0]
    v = vertices.shape[0]
    qp = pl.cdiv(q, QB) * QB
    m2 = jnp.sum(means * means, axis=1, keepdims=True)
    mb = jnp.pad(means.astype(jnp.bfloat16), ((0, qp - q), (0, 0)))
    m2p = jnp.pad(m2, ((0, qp - q), (0, 0)))
    vp = jnp.pad(
        vertices, ((0, VPAD - v), (0, 0)), constant_values=PAD_COORD
    )
    v2 = jnp.sum(vp * vp, axis=1)[None, :]
    vb = vp.astype(jnp.bfloat16).T
    grid = qp // QB
    out = pl.pallas_call(
        _knn_block,
        grid=(grid,),
        in_specs=[
            pl.BlockSpec((QB, 3), lambda i: (i, 0)),
            pl.BlockSpec((QB, 1), lambda i: (i, 0)),
            pl.BlockSpec((3, VPAD), lambda i: (0, 0)),
            pl.BlockSpec((1, VPAD), lambda i: (0, 0)),
        ],
        out_specs=pl.BlockSpec((QB, K), lambda i: (i, 0)),
        out_shape=jax.ShapeDtypeStruct((qp, K), jnp.int32),
    )(mb, m2p, vb, v2)
    return out[:q, :]


def _sc_call(means, vertices):
    q = means.shape[0]
    v = vertices.shape[0]
    qw = pl.cdiv(q, NW * 8) * 8
    qpad = qw * NW
    groups = VPAD // (16 * CPG)

    m2 = jnp.sum(means * means, axis=1, keepdims=True)
    mrb = _round_bf16_f32(means)
    mdat = jnp.pad(
        jnp.concatenate([mrb, m2], axis=1), ((0, qpad - q), (0, 12))
    )
    vp = jnp.pad(
        vertices, ((0, VPAD - v), (0, 0)), constant_values=PAD_COORD
    )
    v2 = jnp.sum(vp * vp, axis=1, keepdims=True)
    vdat = jnp.concatenate([_round_bf16_f32(vp), v2], axis=1).T

    mesh = plsc.VectorSubcoreMesh(core_axis_name="c", subcore_axis_name="s")

    @functools.partial(
        pl.kernel,
        mesh=mesh,
        compiler_params=pltpu.CompilerParams(needs_layout_passes=False, use_tc_tiling_on_sc=False),
        out_type=jax.ShapeDtypeStruct((qpad, 16), jnp.int32),
        scratch_types=[
            pltpu.VMEM((VPAD,), jnp.float32),
            pltpu.VMEM((VPAD,), jnp.float32),
            pltpu.VMEM((VPAD,), jnp.float32),
            pltpu.VMEM((VPAD,), jnp.float32),
            pltpu.VMEM((qw, 16), jnp.float32),
            pltpu.VMEM((qw, 16), jnp.int32),
            pltpu.VMEM((QT, VPAD), jnp.float32),
            pltpu.VMEM((QT, VPAD // CPG), jnp.float32),
            pltpu.VMEM((QT, 16), jnp.float32),
            pltpu.VMEM((QT, 16), jnp.int32),
        ],
    )
    def sc_knn(mdat_h, vdat_h, out_h, vx_v, vy_v, vz_v, v2_v, md_v, out_v,
               sc_ref, gmc_ref, bd_ref, bi_ref):
        cid = lax.axis_index("c")
        sid = lax.axis_index("s")
        wid = sid * 2 + cid
        base = wid * qw
        pltpu.sync_copy(vdat_h.at[0], vx_v)
        pltpu.sync_copy(vdat_h.at[1], vy_v)
        pltpu.sync_copy(vdat_h.at[2], vz_v)
        pltpu.sync_copy(vdat_h.at[3], v2_v)
        pltpu.sync_copy(mdat_h.at[pl.ds(base, qw)], md_v)

        lane = lax.iota(jnp.int32, 16)
        bigv = jnp.full((16,), jnp.float32(BIG))

        gdn = lax.GatherDimensionNumbers(
            offset_dims=(), collapsed_slice_dims=(0,), start_index_map=(0,)
        )

        def shuffle(y, perm):
            return lax.gather(
                y, perm[:, None], gdn, (1,),
                mode=lax.GatherScatterMode.PROMISE_IN_BOUNDS,
            )

        def any_neg(x):
            return plsc.all_reduce_population_count(
                x < jnp.float32(0.0)
            )[0] > 0

        def merge(j, dc, ic):
            bd = bd_ref[j]
            bi = bi_ref[j]
            sd, si = plsc.sort_key_val(dc, ic)
            rd = jnp.flip(sd)
            ri = jnp.flip(si)
            keep = bd <= rd
            lod = jnp.where(keep, bd, rd)
            loi = jnp.where(keep, bi, ri)
            nd, ni = plsc.sort_key_val(lod, loi)
            bd_ref[j] = nd
            bi_ref[j] = ni

        def per_quad(qt, _):
            q0 = qt * QT
            mcoef = []
            for j in range(QT):
                mrow = md_v[q0 + j]
                mcoef.append((mrow[0], mrow[1], mrow[2], mrow[3]))

            def pre_body(g, gms):
                goff = g * (16 * CPG)
                ggm = [None] * QT
                for cc in range(CPG):
                    off = goff + cc * 16
                    vx = vx_v[pl.ds(off, 16)]
                    vy = vy_v[pl.ds(off, 16)]
                    vz = vz_v[pl.ds(off, 16)]
                    v2c = v2_v[pl.ds(off, 16)]
                    for j in range(QT):
                        mxq, myq, mzq, m2q = mcoef[j]
                        mv = (mxq * vx + myq * vy) + mzq * vz
                        dq = (m2q - 2.0 * mv) + v2c
                        sc_ref[j, pl.ds(off, 16)] = dq
                        ggm[j] = (
                            dq if ggm[j] is None
                            else jnp.minimum(ggm[j], dq)
                        )
                out = []
                for j in range(QT):
                    gmc_ref[j, pl.ds(g * 16, 16)] = ggm[j]
                    out.append(jnp.minimum(gms[j], ggm[j]))
                return tuple(out)

            gms = lax.fori_loop(0, groups, pre_body, (bigv,) * QT)

            tau0v = []
            for j in range(QT):
                sk, _ = plsc.sort_key_val(gms[j], lane)
                tau0v.append(jnp.full((16,), sk[8]))
                bd_ref[j] = bigv
                bi_ref[j] = jnp.zeros((16,), jnp.int32)

            def group_body(g, carry):
                goff = g * (16 * CPG)
                gmv = [
                    gmc_ref[j, pl.ds(g * 16, 16)] for j in range(QT)
                ]
                e = gmv[0] - tau0v[0]
                for j in range(1, QT):
                    e = jnp.minimum(e, gmv[j] - tau0v[j])

                @pl.when(any_neg(e))
                def _slow():
                    for j in range(QT):

                        @pl.when(any_neg(gmv[j] - tau0v[j]))
                        def _q(j=j):

                            def tree(excl):
                                m = None
                                mi = None
                                for cc in range(CPG):
                                    ic = lane + (goff + cc * 16)
                                    dc = sc_ref[j, pl.ds(goff + cc * 16, 16)]
                                    if excl is not None:
                                        dc = jnp.where(
                                            dc <= excl, jnp.float32(BIG), dc
                                        )
                                    if m is None:
                                        m, mi = dc, ic
                                    else:
                                        cnd = dc < m
                                        m = jnp.where(cnd, dc, m)
                                        mi = jnp.where(cnd, ic, mi)
                                return m, mi

                            def rounds(r, excl):
                                m, mi = tree(excl)
                                if excl is None:
                                    merge(j, m, mi)
                                    rounds(r - 1, m)
                                else:

                                    @pl.when(
                                        any_neg(m - tau0v[j])
                                    )
                                    def _more():
                                        merge(j, m, mi)
                                        if r > 0:
                                            rounds(r - 1, m)

                            rounds(3, None)

                return carry

            lax.fori_loop(0, groups, group_body, 0)
            for j in range(QT):
                out_v[q0 + j] = bi_ref[j]
            return 0

        lax.fori_loop(0, qw // QT, per_quad, 0)
        pltpu.sync_copy(out_v, out_h.at[pl.ds(base, qw)])

    out = sc_knn(mdat, vdat)
    return out[:q, :K]


def kernel(means, vertices):
    q = means.shape[0]
    if SPLIT <= 0:
        out = _sc_call(means, vertices)
    elif SPLIT >= q:
        out = _tc_call(means, vertices)
    else:
        out_tc = _tc_call(means[:SPLIT], vertices)
        out_sc = _sc_call(means[SPLIT:], vertices)
        out = jnp.concatenate([out_tc, out_sc], axis=0)
    return out, jnp.float32(0.0)


# speedup vs baseline: 4.8716x; 4.8716x over previous
import functools

import jax
import jax.numpy as jnp
from jax import lax
from jax.experimental import pallas as pl
from jax.experimental.pallas import tpu as pltpu
from jax.experimental.pallas import tpu_sc as plsc

K = 8
PAD_COORD = 1.0e18
BIG = 3.0e38

QB = 512
VPAD = 5120

NW = 32
CPG = 8
QT = 4
SPLIT = 37120


def _round_bf16_f32(x):
    u = lax.bitcast_convert_type(x, jnp.uint32)
    r = (u + jnp.uint32(0x7FFF) + ((u >> 16) & jnp.uint32(1))) & jnp.uint32(
        0xFFFF0000
    )
    return lax.bitcast_convert_type(r, jnp.float32)


def _knn_block(mb_ref, m2_ref, vb_ref, v2_ref, out_ref):
    mb = mb_ref[:, :].astype(jnp.float32)
    mx = mb[:, 0:1]
    my = mb[:, 1:2]
    mz = mb[:, 2:3]
    m2 = m2_ref[:, :]
    vb = vb_ref[:, :].astype(jnp.float32)
    vx = vb[0:1, :]
    vy = vb[1:2, :]
    vz = vb[2:3, :]
    v2 = v2_ref[:, :]
    mv = (mx * vx + my * vy) + mz * vz
    d2 = (m2 - 2.0 * mv) + v2
    iot = jax.lax.broadcasted_iota(jnp.int32, (QB, VPAD), 1)
    cols = []
    for _ in range(K):
        mn = jnp.min(d2, axis=1, keepdims=True)
        am = jnp.min(
            jnp.where(d2 == mn, iot, jnp.int32(2**30)), axis=1, keepdims=True
        )
        cols.append(am)
        d2 = jnp.where(iot == am, jnp.float32(jnp.inf), d2)
    out_ref[:, :] = jnp.concatenate(cols, axis=1)


def _tc_call(means, vertices):
    q = means.shape[0]
    v = vertices.shape[0]
    qp = pl.cdiv(q, QB) * QB
    m2 = jnp.sum(means * means, axis=1, keepdims=True)
    mb = jnp.pad(means.astype(jnp.bfloat16), ((0, qp - q), (0, 0)))
    m2p = jnp.pad(m2, ((0, qp - q), (0, 0)))
    vp = jnp.pad(
        vertices, ((0, VPAD - v), (0, 0)), constant_values=PAD_COORD
    )
    v2 = jnp.sum(vp * vp, axis=1)[None, :]
    vb = vp.astype(jnp.bfloat16).T
    grid = qp // QB
    out = pl.pallas_call(
        _knn_block,
        grid=(grid,),
        in_specs=[
            pl.BlockSpec((QB, 3), lambda i: (i, 0)),
            pl.BlockSpec((QB, 1), lambda i: (i, 0)),
            pl.BlockSpec((3, VPAD), lambda i: (0, 0)),
            pl.BlockSpec((1, VPAD), lambda i: (0, 0)),
        ],
        out_specs=pl.BlockSpec((QB, K), lambda i: (i, 0)),
        out_shape=jax.ShapeDtypeStruct((qp, K), jnp.int32),
    )(mb, m2p, vb, v2)
    return out[:q, :]


def _sc_call(means, vertices):
    q = means.shape[0]
    v = vertices.shape[0]
    qw = pl.cdiv(q, NW * 8) * 8
    qpad = qw * NW
    groups = VPAD // (16 * CPG)

    m2 = jnp.sum(means * means, axis=1, keepdims=True)
    mrb = _round_bf16_f32(means)
    mdat = jnp.pad(
        jnp.concatenate([mrb, m2], axis=1), ((0, qpad - q), (0, 12))
    )
    vp = jnp.pad(
        vertices, ((0, VPAD - v), (0, 0)), constant_values=PAD_COORD
    )
    v2 = jnp.sum(vp * vp, axis=1, keepdims=True)
    vdat = jnp.concatenate([_round_bf16_f32(vp), v2], axis=1).T

    mesh = plsc.VectorSubcoreMesh(core_axis_name="c", subcore_axis_name="s")

    @functools.partial(
        pl.kernel,
        mesh=mesh,
        compiler_params=pltpu.CompilerParams(needs_layout_passes=False, use_tc_tiling_on_sc=False),
        out_type=jax.ShapeDtypeStruct((qpad, 16), jnp.int32),
        scratch_types=[
            pltpu.VMEM((VPAD,), jnp.float32),
            pltpu.VMEM((VPAD,), jnp.float32),
            pltpu.VMEM((VPAD,), jnp.float32),
            pltpu.VMEM((VPAD,), jnp.float32),
            pltpu.VMEM((qw, 16), jnp.float32),
            pltpu.VMEM((qw, 16), jnp.int32),
            pltpu.VMEM((QT, VPAD), jnp.float32),
            pltpu.VMEM((QT, VPAD // CPG), jnp.float32),
            pltpu.VMEM((QT, 16), jnp.float32),
            pltpu.VMEM((QT, 16), jnp.int32),
        ],
    )
    def sc_knn(mdat_h, vdat_h, out_h, vx_v, vy_v, vz_v, v2_v, md_v, out_v,
               sc_ref, gmc_ref, bd_ref, bi_ref):
        cid = lax.axis_index("c")
        sid = lax.axis_index("s")
        wid = sid * 2 + cid
        base = wid * qw
        pltpu.sync_copy(vdat_h.at[0], vx_v)
        pltpu.sync_copy(vdat_h.at[1], vy_v)
        pltpu.sync_copy(vdat_h.at[2], vz_v)
        pltpu.sync_copy(vdat_h.at[3], v2_v)
        pltpu.sync_copy(mdat_h.at[pl.ds(base, qw)], md_v)

        lane = lax.iota(jnp.int32, 16)
        bigv = jnp.full((16,), jnp.float32(BIG))

        gdn = lax.GatherDimensionNumbers(
            offset_dims=(), collapsed_slice_dims=(0,), start_index_map=(0,)
        )

        def shuffle(y, perm):
            return lax.gather(
                y, perm[:, None], gdn, (1,),
                mode=lax.GatherScatterMode.PROMISE_IN_BOUNDS,
            )

        def any_neg(x):
            return plsc.all_reduce_population_count(
                x < jnp.float32(0.0)
            )[0] > 0

        def merge(j, dc, ic):
            bd = bd_ref[j]
            bi = bi_ref[j]
            sd, si = plsc.sort_key_val(dc, ic)
            rd = jnp.flip(sd)
            ri = jnp.flip(si)
            keep = bd <= rd
            lod = jnp.where(keep, bd, rd)
            loi = jnp.where(keep, bi, ri)
            nd, ni = plsc.sort_key_val(lod, loi)
            bd_ref[j] = nd
            bi_ref[j] = ni

        def per_quad(qt, _):
            q0 = qt * QT
            mcoef = []
            for j in range(QT):
                mrow = md_v[q0 + j]
                mcoef.append((mrow[0], mrow[1], mrow[2], mrow[3]))

            def pre_body(g, gms):
                goff = g * (16 * CPG)
                ggm = [None] * QT
                for cc in range(CPG):
                    off = goff + cc * 16
                    vx = vx_v[pl.ds(off, 16)]
                    vy = vy_v[pl.ds(off, 16)]
                    vz = vz_v[pl.ds(off, 16)]
                    v2c = v2_v[pl.ds(off, 16)]
                    for j in range(QT):
                        mxq, myq, mzq, m2q = mcoef[j]
                        mv = (mxq * vx + myq * vy) + mzq * vz
                        dq = (m2q - 2.0 * mv) + v2c
                        sc_ref[j, pl.ds(off, 16)] = dq
                        ggm[j] = (
                            dq if ggm[j] is None
                            else jnp.minimum(ggm[j], dq)
                        )
                out = []
                for j in range(QT):
                    gmc_ref[j, pl.ds(g * 16, 16)] = ggm[j]
                    out.append(jnp.minimum(gms[j], ggm[j]))
                return tuple(out)

            gms = lax.fori_loop(0, groups, pre_body, (bigv,) * QT)

            tau0v = []
            for j in range(QT):
                sk, _ = plsc.sort_key_val(gms[j], lane)
                tau0v.append(jnp.full((16,), sk[8]))
                bd_ref[j] = bigv
                bi_ref[j] = jnp.zeros((16,), jnp.int32)

            def group_body(g, carry):
                goff = g * (16 * CPG)
                gmv = [
                    gmc_ref[j, pl.ds(g * 16, 16)] for j in range(QT)
                ]
                e = gmv[0] - tau0v[0]
                for j in range(1, QT):
                    e = jnp.minimum(e, gmv[j] - tau0v[j])

                @pl.when(any_neg(e))
                def _slow():
                    for j in range(QT):

                        @pl.when(any_neg(gmv[j] - tau0v[j]))
                        def _q(j=j):

                            def tree(excl):
                                m = None
                                mi = None
                                for cc in range(CPG):
                                    ic = lane + (goff + cc * 16)
                                    dc = sc_ref[j, pl.ds(goff + cc * 16, 16)]
                                    if excl is not None:
                                        dc = jnp.where(
                                            dc <= excl, jnp.float32(BIG), dc
                                        )
                                    if m is None:
                                        m, mi = dc, ic
                                    else:
                                        cnd = dc < m
                                        m = jnp.where(cnd, dc, m)
                                        mi = jnp.where(cnd, ic, mi)
                                return m, mi

                            def rounds(r, excl):
                                m, mi = tree(excl)
                                if excl is None:
                                    merge(j, m, mi)
                                    rounds(r - 1, m)
                                else:

                                    @pl.when(
                                        any_neg(m - tau0v[j])
                                    )
                                    def _more():
                                        merge(j, m, mi)
                                        if r > 0:
                                            rounds(r - 1, m)

                            rounds(3, None)

                return carry

            lax.fori_loop(0, groups, group_body, 0)
            for j in range(QT):
                out_v[q0 + j] = bi_ref[j]
            return 0

        lax.fori_loop(0, qw // QT, per_quad, 0)
        pltpu.sync_copy(out_v, out_h.at[pl.ds(base, qw)])

    out = sc_knn(mdat, vdat)
    return out[:q, :K]


def kernel(means, vertices):
    q = means.shape[0]
    if SPLIT <= 0:
        out = _sc_call(means, vertices)
    elif SPLIT >= q:
        out = _tc_call(means, vertices)
    else:
        out_tc = _tc_call(means[:SPLIT], vertices)
        out_sc = _sc_call(means[SPLIT:], vertices)
        out = jnp.concatenate([out_tc, out_sc], axis=0)
    return out, jnp.float32(0.0)
